# baseline (device time: 13049 ns/iter reference)
import jax
import jax.numpy as jnp
from jax import lax
from jax.experimental import pallas as pl
from jax.experimental.pallas import tpu as pltpu

N_DEV = 4


def kernel(x, Wg, Wu, Wd):
    m, _ = x.shape
    _, h = Wg.shape

    def body(x_hbm, wg_hbm, wu_hbm, wd_hbm, out_hbm,
             xv, wgv, wuv, wdv, outv, comm_ref,
             in_sems, out_sem, send_sems, recv_sems):
        my_pos = lax.axis_index("i")

        cp_x = pltpu.make_async_copy(x_hbm, xv, in_sems.at[0])
        cp_g = pltpu.make_async_copy(wg_hbm, wgv, in_sems.at[1])
        cp_u = pltpu.make_async_copy(wu_hbm, wuv, in_sems.at[2])
        cp_d = pltpu.make_async_copy(wd_hbm, wdv, in_sems.at[3])
        cp_x.start()
        cp_g.start()
        cp_u.start()
        cp_d.start()

        barrier_sem = pltpu.get_barrier_semaphore()
        for d in range(1, N_DEV):
            pl.semaphore_signal(
                barrier_sem, inc=1,
                device_id=((my_pos + d) % N_DEV,),
                device_id_type=pl.DeviceIdType.MESH,
            )

        cp_x.wait()
        cp_g.wait()
        cp_u.wait()
        xb = xv[:, :].astype(jnp.bfloat16)
        gate = jnp.dot(xb, wgv[:, :].astype(jnp.bfloat16),
                       preferred_element_type=jnp.float32)
        up = jnp.dot(xb, wuv[:, :].astype(jnp.bfloat16),
                     preferred_element_type=jnp.float32)
        hidden = gate * (up * jax.nn.sigmoid(up))
        cp_d.wait()
        partial = jnp.dot(hidden.astype(jnp.bfloat16),
                          wdv[:, :].astype(jnp.bfloat16),
                          preferred_element_type=jnp.float32)
        comm_ref[0, :, :] = partial.astype(jnp.bfloat16)

        pl.semaphore_wait(barrier_sem, N_DEV - 1)

        rdmas = []
        for d in range(1, N_DEV):
            rdma = pltpu.make_async_remote_copy(
                src_ref=comm_ref.at[0],
                dst_ref=comm_ref.at[d],
                send_sem=send_sems.at[d - 1],
                recv_sem=recv_sems.at[d - 1],
                device_id=((my_pos + d) % N_DEV,),
                device_id_type=pl.DeviceIdType.MESH,
            )
            rdma.start()
            rdmas.append(rdma)

        for rdma in rdmas:
            rdma.wait_recv()

        acc = partial
        for d in range(1, N_DEV):
            acc = acc + comm_ref[d, :, :].astype(jnp.float32)
        outv[:, :] = acc.astype(jnp.bfloat16)

        cp_out = pltpu.make_async_copy(outv, out_hbm, out_sem)
        cp_out.start()
        cp_out.wait()

        for rdma in rdmas:
            rdma.wait_send()

    return pl.pallas_call(
        body,
        out_shape=jax.ShapeDtypeStruct((m, m), jnp.bfloat16),
        in_specs=[pl.BlockSpec(memory_space=pl.ANY)] * 4,
        out_specs=pl.BlockSpec(memory_space=pl.ANY),
        scratch_shapes=[
            pltpu.VMEM((m, m), jnp.float32),
            pltpu.VMEM((m, h), jnp.float32),
            pltpu.VMEM((m, h), jnp.float32),
            pltpu.VMEM((h, m), jnp.float32),
            pltpu.VMEM((m, m), jnp.bfloat16),
            pltpu.VMEM((N_DEV, m, m), jnp.bfloat16),
            pltpu.SemaphoreType.DMA((4,)),
            pltpu.SemaphoreType.DMA,
            pltpu.SemaphoreType.DMA((N_DEV - 1,)),
            pltpu.SemaphoreType.DMA((N_DEV - 1,)),
        ],
        compiler_params=pltpu.CompilerParams(collective_id=0),
    )(x, Wg, Wu, Wd)


# device time: 11743 ns/iter; 1.1112x vs baseline; 1.1112x over previous
import jax
import jax.numpy as jnp
from jax import lax
from jax.experimental import pallas as pl
from jax.experimental.pallas import tpu as pltpu

N_DEV = 4


def kernel(x, Wg, Wu, Wd):
    m, _ = x.shape

    def body(x_ref, wg_ref, wu_ref, wd_ref, out_ref, comm_ref,
             send_sems, recv_sems):
        my_pos = lax.axis_index("i")

        barrier_sem = pltpu.get_barrier_semaphore()
        for d in range(1, N_DEV):
            pl.semaphore_signal(
                barrier_sem, inc=1,
                device_id=((my_pos + d) % N_DEV,),
                device_id_type=pl.DeviceIdType.MESH,
            )

        xb = x_ref[:, :]
        gate = jnp.dot(xb, wg_ref[:, :], preferred_element_type=jnp.float32)
        up = jnp.dot(xb, wu_ref[:, :], preferred_element_type=jnp.float32)
        hidden = gate * (up * jax.nn.sigmoid(up))
        partial = jnp.dot(hidden.astype(jnp.bfloat16), wd_ref[:, :],
                          preferred_element_type=jnp.float32)
        comm_ref[0, :, :] = partial.astype(jnp.bfloat16)

        pl.semaphore_wait(barrier_sem, N_DEV - 1)

        rdmas = []
        for d in range(1, N_DEV):
            rdma = pltpu.make_async_remote_copy(
                src_ref=comm_ref.at[0],
                dst_ref=comm_ref.at[d],
                send_sem=send_sems.at[d - 1],
                recv_sem=recv_sems.at[d - 1],
                device_id=((my_pos + d) % N_DEV,),
                device_id_type=pl.DeviceIdType.MESH,
            )
            rdma.start()
            rdmas.append(rdma)

        for rdma in rdmas:
            rdma.wait_recv()

        acc = partial
        for d in range(1, N_DEV):
            acc = acc + comm_ref[d, :, :].astype(jnp.float32)
        out_ref[:, :] = acc.astype(jnp.bfloat16)

        for rdma in rdmas:
            rdma.wait_send()

    return pl.pallas_call(
        body,
        out_shape=jax.ShapeDtypeStruct((m, m), jnp.bfloat16),
        in_specs=[pl.BlockSpec(memory_space=pltpu.VMEM)] * 4,
        out_specs=pl.BlockSpec(memory_space=pltpu.VMEM),
        scratch_shapes=[
            pltpu.VMEM((N_DEV, m, m), jnp.bfloat16),
            pltpu.SemaphoreType.DMA((N_DEV - 1,)),
            pltpu.SemaphoreType.DMA((N_DEV - 1,)),
        ],
        compiler_params=pltpu.CompilerParams(collective_id=0),
    )(
        x.astype(jnp.bfloat16),
        Wg.astype(jnp.bfloat16),
        Wu.astype(jnp.bfloat16),
        Wd.astype(jnp.bfloat16),
    )
